# baseline (speedup 1.0000x reference)
import numpy as np
import jax
import jax.numpy as jnp
from jax import lax
from jax.experimental import pallas as pl
from jax.experimental.pallas import tpu as pltpu

C1, C2, C3 = 16, 16, 10
NB = 256

_K0 = [128 * (max(2 * oh - 1, 0) * 28 // 128) for oh in range(14)]
_KW = [min(256, 784 - k0) for k0 in _K0]


def _conv1_onehot():
    e = np.zeros((9, 14, 256, 14), np.float32)
    for kh in range(3):
        for kw in range(3):
            for oh in range(14):
                r = 2 * oh + kh - 1
                if not 0 <= r < 28:
                    continue
                for ow in range(14):
                    c = 2 * ow + kw - 1
                    if 0 <= c < 28:
                        e[kh * 3 + kw, oh, r * 28 + c - _K0[oh], ow] = 1.0
    return e.reshape(9, 14 * 256 * 14)


def _col_onehot(n_out, n_in, cpad):
    p = np.zeros((3, n_out, cpad), np.float32)
    for kw in range(3):
        for ow in range(n_out):
            c = 2 * ow + kw - 1
            if 0 <= c < n_in:
                p[kw, ow, c] = 1.0
    return p


_E1 = jnp.asarray(_conv1_onehot(), jnp.bfloat16)
_P2 = jnp.asarray(_col_onehot(7, 14, 16), jnp.bfloat16)
_P3 = jnp.asarray(_col_onehot(4, 7, 8), jnp.bfloat16)
_GP = np.zeros((40, 10), np.float32)
for _q in range(4):
    for _c in range(10):
        _GP[_q * 10 + _c, _c] = 1.0 / 16.0
_GP = jnp.asarray(_GP)


def _body(x_ref, w1w_ref, b1_ref, w2r_ref, b2_ref, w3r_ref, b3_ref, gp_ref,
          o_ref, xb_ref, a_ref, c_ref):
    nb = x_ref.shape[0]
    f32 = jnp.float32
    bf16 = jnp.bfloat16

    xb_ref[...] = x_ref[...].astype(bf16)
    a_ref[...] = jnp.zeros(a_ref.shape, bf16)
    c_ref[...] = jnp.zeros(c_ref.shape, bf16)

    b1b = b1_ref[...]
    for oh in range(14):
        k0, kw = _K0[oh], _KW[oh]
        y = jnp.dot(xb_ref[:, pl.ds(k0, kw)], w1w_ref[oh, pl.ds(0, kw), :],
                    preferred_element_type=f32)
        a_ref[:, pl.ds((oh + 1) * 256, 224)] = \
            jnp.maximum(y + b1b, 0.0).astype(bf16)

    b2b = b2_ref[...]
    w2r = w2r_ref[...]
    for oh in range(7):
        y = jnp.dot(a_ref[:, pl.ds(oh * 512, 768)], w2r,
                    preferred_element_type=f32)
        c_ref[:, pl.ds((oh + 1) * 128, 112)] = \
            jnp.maximum(y + b2b, 0.0).astype(bf16)

    b3b = b3_ref[...]
    w3r = w3r_ref[...]
    s = jnp.zeros((nb, 40), f32)
    for oh in range(4):
        y = jnp.dot(c_ref[:, pl.ds(oh * 256, 384)], w3r,
                    preferred_element_type=f32)
        s = s + jnp.maximum(y + b3b, 0.0)
    o_ref[...] = jnp.dot(s, gp_ref[...], preferred_element_type=f32)


def kernel(x, w1, b1, w2, b2, w3, b3):
    n = x.shape[0]
    nb = NB
    n_pad = -(-n // nb) * nb
    xin = jnp.pad(x, ((0, n_pad - n), (0, 0))) if n_pad != n else x
    bf16 = jnp.bfloat16

    w1w = lax.dot_general(_E1, w1.reshape(16, 9).astype(bf16),
                          (((0,), (1,)), ((), ())),
                          preferred_element_type=jnp.float32)
    w1w = w1w.reshape(14, 256, 224).astype(bf16)

    w2r = jnp.einsum('bahw,woc->hcaob', w2.astype(bf16), _P2,
                     preferred_element_type=jnp.float32)
    w2r = w2r.reshape(768, 112).astype(bf16)
    w3r = jnp.einsum('bahw,woc->hcaob', w3.astype(bf16), _P3,
                     preferred_element_type=jnp.float32)
    w3r = w3r.reshape(384, 40).astype(bf16)

    b1r = jnp.tile(b1, (14,))[None, :]
    b2r = jnp.tile(b2, (7,))[None, :]
    b3r = jnp.tile(b3, (4,))[None, :]

    out = pl.pallas_call(
        _body,
        out_shape=jax.ShapeDtypeStruct((n_pad, C3), jnp.float32),
        grid=(n_pad // nb,),
        in_specs=[
            pl.BlockSpec((nb, 784), lambda i: (i, 0)),
            pl.BlockSpec((14, 256, 224), lambda i: (0, 0, 0)),
            pl.BlockSpec((1, 224), lambda i: (0, 0)),
            pl.BlockSpec((768, 112), lambda i: (0, 0)),
            pl.BlockSpec((1, 112), lambda i: (0, 0)),
            pl.BlockSpec((384, 40), lambda i: (0, 0)),
            pl.BlockSpec((1, 40), lambda i: (0, 0)),
            pl.BlockSpec((40, 10), lambda i: (0, 0)),
        ],
        out_specs=pl.BlockSpec((nb, C3), lambda i: (i, 0)),
        scratch_shapes=[
            pltpu.VMEM((nb, 784), jnp.bfloat16),
            pltpu.VMEM((nb, 15 * 256), jnp.bfloat16),
            pltpu.VMEM((nb, 9 * 128), jnp.bfloat16),
        ],
        compiler_params=pltpu.CompilerParams(
            dimension_semantics=("parallel",),
            vmem_limit_bytes=64 * 1024 * 1024),
    )(xin, w1w, b1r, w2r, b2r, w3r, b3r, _GP)

    return out[:n]


# speedup vs baseline: 1.2286x; 1.2286x over previous
import numpy as np
import jax
import jax.numpy as jnp
from jax.experimental import pallas as pl
from jax.experimental.pallas import tpu as pltpu

C1, C2, C3 = 16, 16, 10
NB = 512

_K0 = [128 * (max(2 * oh - 1, 0) * 28 // 128) for oh in range(14)]
_KW = [min(256, 784 - k0) for k0 in _K0]


def _body(x_ref, w1w_ref, b1_ref, w2r_ref, b2_ref, w3r_ref, b3_ref, gp_ref,
          o_ref, xb_ref, a_ref, c_ref):
    nb = x_ref.shape[0]
    f32 = jnp.float32
    bf16 = jnp.bfloat16

    xb_ref[...] = x_ref[...].astype(bf16)

    a_ref[:, pl.ds(0, 256)] = jnp.zeros((nb, 256), bf16)
    b1b = b1_ref[...]
    for oh in range(14):
        k0, kw = _K0[oh], _KW[oh]
        y = jnp.dot(xb_ref[:, pl.ds(k0, kw)], w1w_ref[oh, pl.ds(0, kw), :],
                    preferred_element_type=f32)
        a_ref[:, pl.ds((oh + 1) * 256, 256)] = \
            jnp.maximum(y + b1b, 0.0).astype(bf16)

    c_ref[:, pl.ds(0, 128)] = jnp.zeros((nb, 128), bf16)
    c_ref[:, pl.ds(8 * 128, 128)] = jnp.zeros((nb, 128), bf16)
    b2b = b2_ref[...]
    w2r = w2r_ref[...]
    for oh in range(7):
        y = jnp.dot(a_ref[:, pl.ds(oh * 512, 768)], w2r,
                    preferred_element_type=f32)
        c_ref[:, pl.ds((oh + 1) * 128, 128)] = \
            jnp.maximum(y + b2b, 0.0).astype(bf16)

    b3b = b3_ref[...]
    w3r = w3r_ref[...]
    s = jnp.zeros((nb, 40), f32)
    for oh in range(4):
        y = jnp.dot(c_ref[:, pl.ds(oh * 256, 384)], w3r,
                    preferred_element_type=f32)
        s = s + jnp.maximum(y + b3b, 0.0)
    o_ref[...] = jnp.dot(s, gp_ref[...], preferred_element_type=f32)


def _build_operators(w1, b1, w2, b2, w3, b3):
    bf16 = jnp.bfloat16

    e1 = np.zeros((3, 3, 14, 256, 14), np.float32)
    for kh in range(3):
        for kw in range(3):
            for oh in range(14):
                r = 2 * oh + kh - 1
                if not 0 <= r < 28:
                    continue
                for ow in range(14):
                    c = 2 * ow + kw - 1
                    if 0 <= c < 28:
                        e1[kh, kw, oh, r * 28 + c - _K0[oh], ow] = 1.0
    w1w = jnp.einsum('ahw,hwokq->okaq', w1[:, 0].astype(bf16),
                     jnp.asarray(e1, bf16),
                     preferred_element_type=jnp.float32)
    w1w = w1w.reshape(14, 256, 224)
    w1w = jnp.pad(w1w, ((0, 0), (0, 0), (0, 32))).astype(bf16)

    p2 = np.zeros((3, 7, 14), np.float32)
    for kw in range(3):
        for ow in range(7):
            c = 2 * ow + kw - 1
            if 0 <= c < 14:
                p2[kw, ow, c] = 1.0
    w2r = jnp.einsum('abhw,woc->hbcao', w2.astype(bf16),
                     jnp.asarray(p2, bf16),
                     preferred_element_type=jnp.float32)
    w2r = w2r.reshape(3, 224, 112)
    w2r = jnp.pad(w2r, ((0, 0), (0, 32), (0, 16))).reshape(768, 128)
    w2r = w2r.astype(bf16)

    p3 = np.zeros((3, 4, 7), np.float32)
    for kw in range(3):
        for ow in range(4):
            c = 2 * ow + kw - 1
            if 0 <= c < 7:
                p3[kw, ow, c] = 1.0
    w3r = jnp.einsum('abhw,woc->hbcao', w3.astype(bf16),
                     jnp.asarray(p3, bf16),
                     preferred_element_type=jnp.float32)
    w3r = w3r.reshape(3, 112, 40)
    w3r = jnp.pad(w3r, ((0, 0), (0, 16), (0, 0))).reshape(384, 40)
    w3r = w3r.astype(bf16)

    b1r = jnp.pad(jnp.repeat(b1, 14), (0, 32))[None, :]
    b2r = jnp.pad(jnp.repeat(b2, 7), (0, 16))[None, :]
    b3r = jnp.repeat(b3, 4)[None, :]

    gp = np.zeros((40, 10), np.float32)
    for ch in range(10):
        gp[ch * 4:(ch + 1) * 4, ch] = 1.0 / 16.0
    return w1w, b1r, w2r, b2r, w3r, b3r, jnp.asarray(gp)


def kernel(x, w1, b1, w2, b2, w3, b3):
    n = x.shape[0]
    nb = NB
    n_pad = -(-n // nb) * nb
    xin = jnp.pad(x, ((0, n_pad - n), (0, 0))) if n_pad != n else x

    w1w, b1r, w2r, b2r, w3r, b3r, gp = _build_operators(w1, b1, w2, b2,
                                                        w3, b3)
    out = pl.pallas_call(
        _body,
        out_shape=jax.ShapeDtypeStruct((n_pad, C3), jnp.float32),
        grid=(n_pad // nb,),
        in_specs=[
            pl.BlockSpec((nb, 784), lambda i: (i, 0)),
            pl.BlockSpec((14, 256, 256), lambda i: (0, 0, 0)),
            pl.BlockSpec((1, 256), lambda i: (0, 0)),
            pl.BlockSpec((768, 128), lambda i: (0, 0)),
            pl.BlockSpec((1, 128), lambda i: (0, 0)),
            pl.BlockSpec((384, 40), lambda i: (0, 0)),
            pl.BlockSpec((1, 40), lambda i: (0, 0)),
            pl.BlockSpec((40, 10), lambda i: (0, 0)),
        ],
        out_specs=pl.BlockSpec((nb, C3), lambda i: (i, 0)),
        scratch_shapes=[
            pltpu.VMEM((nb, 784), jnp.bfloat16),
            pltpu.VMEM((nb, 15 * 256), jnp.bfloat16),
            pltpu.VMEM((nb, 9 * 128), jnp.bfloat16),
        ],
        compiler_params=pltpu.CompilerParams(
            dimension_semantics=("parallel",),
            vmem_limit_bytes=64 * 1024 * 1024),
    )(xin, w1w, b1r, w2r, b2r, w3r, b3r, gp)

    return out[:n]


# speedup vs baseline: 1.2518x; 1.0189x over previous
import numpy as np
import jax
import jax.numpy as jnp
from jax.experimental import pallas as pl
from jax.experimental.pallas import tpu as pltpu

C1, C2, C3 = 16, 16, 10
NB = 1024

_K0 = [128 * (max(2 * oh - 1, 0) * 28 // 128) for oh in range(14)]
_KW = [min(256, 784 - k0) for k0 in _K0]


def _body(x_ref, w1w_ref, b1_ref, w2r_ref, b2_ref, w3r_ref, b3_ref, gp_ref,
          o_ref, xb_ref, a_ref, c_ref):
    nb = x_ref.shape[0]
    f32 = jnp.float32
    bf16 = jnp.bfloat16

    xb_ref[...] = x_ref[...].astype(bf16)

    a_ref[:, pl.ds(0, 256)] = jnp.zeros((nb, 256), bf16)
    b1b = b1_ref[...]
    for oh in range(14):
        k0, kw = _K0[oh], _KW[oh]
        y = jnp.dot(xb_ref[:, pl.ds(k0, kw)], w1w_ref[oh, pl.ds(0, kw), :],
                    preferred_element_type=f32)
        a_ref[:, pl.ds((oh + 1) * 256, 256)] = \
            jnp.maximum(y + b1b, 0.0).astype(bf16)

    c_ref[:, pl.ds(0, 128)] = jnp.zeros((nb, 128), bf16)
    c_ref[:, pl.ds(8 * 128, 128)] = jnp.zeros((nb, 128), bf16)
    b2b = b2_ref[...]
    w2r = w2r_ref[...]
    for oh in range(7):
        y = jnp.dot(a_ref[:, pl.ds(oh * 512, 768)], w2r,
                    preferred_element_type=f32)
        c_ref[:, pl.ds((oh + 1) * 128, 128)] = \
            jnp.maximum(y + b2b, 0.0).astype(bf16)

    b3b = b3_ref[...]
    w3r = w3r_ref[...]
    s = jnp.zeros((nb, 40), f32)
    for oh in range(4):
        y = jnp.dot(c_ref[:, pl.ds(oh * 256, 384)], w3r,
                    preferred_element_type=f32)
        s = s + jnp.maximum(y + b3b, 0.0)
    o_ref[...] = jnp.dot(s, gp_ref[...], preferred_element_type=f32)


def _build_operators(w1, b1, w2, b2, w3, b3):
    bf16 = jnp.bfloat16

    e1 = np.zeros((3, 3, 14, 256, 14), np.float32)
    for kh in range(3):
        for kw in range(3):
            for oh in range(14):
                r = 2 * oh + kh - 1
                if not 0 <= r < 28:
                    continue
                for ow in range(14):
                    c = 2 * ow + kw - 1
                    if 0 <= c < 28:
                        e1[kh, kw, oh, r * 28 + c - _K0[oh], ow] = 1.0
    w1w = jnp.einsum('ahw,hwokq->okaq', w1[:, 0].astype(bf16),
                     jnp.asarray(e1, bf16),
                     preferred_element_type=jnp.float32)
    w1w = w1w.reshape(14, 256, 224)
    w1w = jnp.pad(w1w, ((0, 0), (0, 0), (0, 32))).astype(bf16)

    p2 = np.zeros((3, 7, 14), np.float32)
    for kw in range(3):
        for ow in range(7):
            c = 2 * ow + kw - 1
            if 0 <= c < 14:
                p2[kw, ow, c] = 1.0
    w2r = jnp.einsum('abhw,woc->hbcao', w2.astype(bf16),
                     jnp.asarray(p2, bf16),
                     preferred_element_type=jnp.float32)
    w2r = w2r.reshape(3, 224, 112)
    w2r = jnp.pad(w2r, ((0, 0), (0, 32), (0, 16))).reshape(768, 128)
    w2r = w2r.astype(bf16)

    p3 = np.zeros((3, 4, 7), np.float32)
    for kw in range(3):
        for ow in range(4):
            c = 2 * ow + kw - 1
            if 0 <= c < 7:
                p3[kw, ow, c] = 1.0
    w3r = jnp.einsum('abhw,woc->hbcao', w3.astype(bf16),
                     jnp.asarray(p3, bf16),
                     preferred_element_type=jnp.float32)
    w3r = w3r.reshape(3, 112, 40)
    w3r = jnp.pad(w3r, ((0, 0), (0, 16), (0, 0))).reshape(384, 40)
    w3r = w3r.astype(bf16)

    b1r = jnp.pad(jnp.repeat(b1, 14), (0, 32))[None, :]
    b2r = jnp.pad(jnp.repeat(b2, 7), (0, 16))[None, :]
    b3r = jnp.repeat(b3, 4)[None, :]

    gp = np.zeros((40, 10), np.float32)
    for ch in range(10):
        gp[ch * 4:(ch + 1) * 4, ch] = 1.0 / 16.0
    return w1w, b1r, w2r, b2r, w3r, b3r, jnp.asarray(gp)


def kernel(x, w1, b1, w2, b2, w3, b3):
    n = x.shape[0]
    nb = NB
    n_pad = -(-n // nb) * nb
    xin = jnp.pad(x, ((0, n_pad - n), (0, 0))) if n_pad != n else x

    w1w, b1r, w2r, b2r, w3r, b3r, gp = _build_operators(w1, b1, w2, b2,
                                                        w3, b3)
    out = pl.pallas_call(
        _body,
        out_shape=jax.ShapeDtypeStruct((n_pad, C3), jnp.float32),
        grid=(n_pad // nb,),
        in_specs=[
            pl.BlockSpec((nb, 784), lambda i: (i, 0)),
            pl.BlockSpec((14, 256, 256), lambda i: (0, 0, 0)),
            pl.BlockSpec((1, 256), lambda i: (0, 0)),
            pl.BlockSpec((768, 128), lambda i: (0, 0)),
            pl.BlockSpec((1, 128), lambda i: (0, 0)),
            pl.BlockSpec((384, 40), lambda i: (0, 0)),
            pl.BlockSpec((1, 40), lambda i: (0, 0)),
            pl.BlockSpec((40, 10), lambda i: (0, 0)),
        ],
        out_specs=pl.BlockSpec((nb, C3), lambda i: (i, 0)),
        scratch_shapes=[
            pltpu.VMEM((nb, 784), jnp.bfloat16),
            pltpu.VMEM((nb, 15 * 256), jnp.bfloat16),
            pltpu.VMEM((nb, 9 * 128), jnp.bfloat16),
        ],
        compiler_params=pltpu.CompilerParams(
            dimension_semantics=("parallel",),
            vmem_limit_bytes=64 * 1024 * 1024),
    )(xin, w1w, b1r, w2r, b2r, w3r, b3r, gp)

    return out[:n]


# speedup vs baseline: 1.2525x; 1.0005x over previous
import numpy as np
import jax
import jax.numpy as jnp
from jax.experimental import pallas as pl
from jax.experimental.pallas import tpu as pltpu

C1, C2, C3 = 16, 16, 10
NB = 2048

_K0 = [128 * (max(2 * oh - 1, 0) * 28 // 128) for oh in range(14)]
_KW = [min(256, 784 - k0) for k0 in _K0]


def _body(x_ref, w1w_ref, b1_ref, w2r_ref, b2_ref, w3r_ref, b3_ref, gp_ref,
          o_ref, xb_ref, a_ref, c_ref):
    nb = x_ref.shape[0]
    f32 = jnp.float32
    bf16 = jnp.bfloat16

    xb_ref[...] = x_ref[...].astype(bf16)

    a_ref[:, pl.ds(0, 256)] = jnp.zeros((nb, 256), bf16)
    b1b = b1_ref[...]
    for oh in range(14):
        k0, kw = _K0[oh], _KW[oh]
        y = jnp.dot(xb_ref[:, pl.ds(k0, kw)], w1w_ref[oh, pl.ds(0, kw), :],
                    preferred_element_type=f32)
        a_ref[:, pl.ds((oh + 1) * 256, 256)] = \
            jnp.maximum(y + b1b, 0.0).astype(bf16)

    c_ref[:, pl.ds(0, 128)] = jnp.zeros((nb, 128), bf16)
    c_ref[:, pl.ds(8 * 128, 128)] = jnp.zeros((nb, 128), bf16)
    b2b = b2_ref[...]
    w2r = w2r_ref[...]
    for oh in range(7):
        y = jnp.dot(a_ref[:, pl.ds(oh * 512, 768)], w2r,
                    preferred_element_type=f32)
        c_ref[:, pl.ds((oh + 1) * 128, 128)] = \
            jnp.maximum(y + b2b, 0.0).astype(bf16)

    b3b = b3_ref[...]
    w3r = w3r_ref[...]
    s = jnp.zeros((nb, 40), f32)
    for oh in range(4):
        y = jnp.dot(c_ref[:, pl.ds(oh * 256, 384)], w3r,
                    preferred_element_type=f32)
        s = s + jnp.maximum(y + b3b, 0.0)
    o_ref[...] = jnp.dot(s, gp_ref[...], preferred_element_type=f32)


def _build_operators(w1, b1, w2, b2, w3, b3):
    bf16 = jnp.bfloat16

    e1 = np.zeros((3, 3, 14, 256, 14), np.float32)
    for kh in range(3):
        for kw in range(3):
            for oh in range(14):
                r = 2 * oh + kh - 1
                if not 0 <= r < 28:
                    continue
                for ow in range(14):
                    c = 2 * ow + kw - 1
                    if 0 <= c < 28:
                        e1[kh, kw, oh, r * 28 + c - _K0[oh], ow] = 1.0
    w1w = jnp.einsum('ahw,hwokq->okaq', w1[:, 0].astype(bf16),
                     jnp.asarray(e1, bf16),
                     preferred_element_type=jnp.float32)
    w1w = w1w.reshape(14, 256, 224)
    w1w = jnp.pad(w1w, ((0, 0), (0, 0), (0, 32))).astype(bf16)

    p2 = np.zeros((3, 7, 14), np.float32)
    for kw in range(3):
        for ow in range(7):
            c = 2 * ow + kw - 1
            if 0 <= c < 14:
                p2[kw, ow, c] = 1.0
    w2r = jnp.einsum('abhw,woc->hbcao', w2.astype(bf16),
                     jnp.asarray(p2, bf16),
                     preferred_element_type=jnp.float32)
    w2r = w2r.reshape(3, 224, 112)
    w2r = jnp.pad(w2r, ((0, 0), (0, 32), (0, 16))).reshape(768, 128)
    w2r = w2r.astype(bf16)

    p3 = np.zeros((3, 4, 7), np.float32)
    for kw in range(3):
        for ow in range(4):
            c = 2 * ow + kw - 1
            if 0 <= c < 7:
                p3[kw, ow, c] = 1.0
    w3r = jnp.einsum('abhw,woc->hbcao', w3.astype(bf16),
                     jnp.asarray(p3, bf16),
                     preferred_element_type=jnp.float32)
    w3r = w3r.reshape(3, 112, 40)
    w3r = jnp.pad(w3r, ((0, 0), (0, 16), (0, 0))).reshape(384, 40)
    w3r = w3r.astype(bf16)

    b1r = jnp.pad(jnp.repeat(b1, 14), (0, 32))[None, :]
    b2r = jnp.pad(jnp.repeat(b2, 7), (0, 16))[None, :]
    b3r = jnp.repeat(b3, 4)[None, :]

    gp = np.zeros((40, 10), np.float32)
    for ch in range(10):
        gp[ch * 4:(ch + 1) * 4, ch] = 1.0 / 16.0
    return w1w, b1r, w2r, b2r, w3r, b3r, jnp.asarray(gp)


def kernel(x, w1, b1, w2, b2, w3, b3):
    n = x.shape[0]
    nb = NB
    n_pad = -(-n // nb) * nb
    xin = jnp.pad(x, ((0, n_pad - n), (0, 0))) if n_pad != n else x

    w1w, b1r, w2r, b2r, w3r, b3r, gp = _build_operators(w1, b1, w2, b2,
                                                        w3, b3)
    out = pl.pallas_call(
        _body,
        out_shape=jax.ShapeDtypeStruct((n_pad, C3), jnp.float32),
        grid=(n_pad // nb,),
        in_specs=[
            pl.BlockSpec((nb, 784), lambda i: (i, 0)),
            pl.BlockSpec((14, 256, 256), lambda i: (0, 0, 0)),
            pl.BlockSpec((1, 256), lambda i: (0, 0)),
            pl.BlockSpec((768, 128), lambda i: (0, 0)),
            pl.BlockSpec((1, 128), lambda i: (0, 0)),
            pl.BlockSpec((384, 40), lambda i: (0, 0)),
            pl.BlockSpec((1, 40), lambda i: (0, 0)),
            pl.BlockSpec((40, 10), lambda i: (0, 0)),
        ],
        out_specs=pl.BlockSpec((nb, C3), lambda i: (i, 0)),
        scratch_shapes=[
            pltpu.VMEM((nb, 784), jnp.bfloat16),
            pltpu.VMEM((nb, 15 * 256), jnp.bfloat16),
            pltpu.VMEM((nb, 9 * 128), jnp.bfloat16),
        ],
        compiler_params=pltpu.CompilerParams(
            dimension_semantics=("parallel",),
            vmem_limit_bytes=64 * 1024 * 1024),
    )(xin, w1w, b1r, w2r, b2r, w3r, b3r, gp)

    return out[:n]
